# baseline (speedup 1.0000x reference)
import functools

import jax
import jax.numpy as jnp
from jax import lax
from jax.experimental import pallas as pl
from jax.experimental.pallas import tpu as pltpu
from jax.experimental.pallas import tpu_sc as plsc

NC = 2
NS = 16
L = 16
NW = NC * NS

N = 4096
C = 4096
RPW = N // NW
RB = 8
NB = RPW // RB

_CP = pltpu.CompilerParams(needs_layout_passes=False)


def _worker_id():
    return lax.axis_index("s") * NC + lax.axis_index("c")


_mesh = plsc.VectorSubcoreMesh(core_axis_name="c", subcore_axis_name="s")


_RBLK = 256


def _den_tc_body(thr_ref, x_ref, den_ref):
    x = x_ref[...]
    acc = jnp.zeros(x.shape, jnp.int32)
    for m in range(7):
        acc += (x >= thr_ref[m]).astype(jnp.int32)
    den_ref[...] = jnp.sum(acc, axis=1).astype(jnp.float32)


def _den_tc(x, thr):
    return pl.pallas_call(
        _den_tc_body,
        grid=(N // _RBLK,),
        in_specs=[
            pl.BlockSpec(memory_space=pltpu.SMEM),
            pl.BlockSpec((_RBLK, C), lambda i: (i, 0)),
        ],
        out_specs=pl.BlockSpec((_RBLK,), lambda i: (i,)),
        out_shape=jax.ShapeDtypeStruct((N,), jnp.float32),
    )(thr, x)


WPB = RB // 4
BLK = 8
BPB = BLK // WPB


@functools.partial(
    pl.kernel,
    out_type=jax.ShapeDtypeStruct((N, C), jnp.uint8),
    mesh=_mesh,
    compiler_params=_CP,
    scratch_types=[
        pltpu.VMEM((2, RB, C), jnp.int32),
        pltpu.VMEM((256,), jnp.int32),
        pltpu.VMEM((256 * L,), jnp.float32),
        pltpu.VMEM((N,), jnp.float32),
        pltpu.VMEM((BLK, C), jnp.int32),
        pltpu.SemaphoreType.DMA,
        pltpu.SemaphoreType.DMA,
        pltpu.SemaphoreType.DMA,
    ],
)
def _main_kernel(x_hbm, ntab_hbm, den_hbm, out_hbm,
                 xbuf, ntab, ntab_f, den, outbuf,
                 sin0, sin1, sout):
    wid = _worker_id()
    base_row = wid * RPW
    wout = out_hbm.bitcast(jnp.int32)
    base_wr = wid * (RPW // 4)
    pltpu.sync_copy(ntab_hbm, ntab)
    pltpu.sync_copy(den_hbm, den)
    sin = (sin0, sin1)
    lanes = lax.iota(jnp.int32, L)

    @plsc.parallel_loop(0, 256, unroll=4)
    def _(v):
        nv = plsc.load_gather(ntab, [jnp.full((L,), v, jnp.int32)])
        ntab_f[pl.ds(v * L, L)] = nv.astype(jnp.float32)

    def in_src(b):
        return x_hbm.at[pl.ds(base_row + b * RB, RB)]

    def out_dst(blk):
        return wout.at[pl.ds(base_wr + blk * BLK, BLK)]

    pltpu.async_copy(in_src(0), xbuf.at[0], sin0)

    for b in range(NB):
        s = b & 1
        blk, bi = divmod(b, BPB)
        pltpu.make_async_copy(in_src(b), xbuf.at[s], sin[s]).wait()
        if b + 1 < NB:
            pltpu.async_copy(in_src(b + 1), xbuf.at[1 - s], sin[1 - s])
        if bi == 0 and blk > 0:
            pltpu.make_async_copy(outbuf, out_dst(blk - 1), sout).wait()
        xb = xbuf.at[s]

        @plsc.parallel_loop(0, WPB * (C // L), unroll=2)
        def _(t):
            wr = t >> 8
            cg = t & (C // L - 1)
            dv = den[pl.ds(cg * L, L)]
            word = jnp.zeros((L,), jnp.int32)
            for r in range(4):
                xv = xb[wr * 4 + r, pl.ds(cg * L, L)]
                num = plsc.load_gather(ntab_f, [(xv << 4) | lanes])
                y = jnp.minimum(num / dv, 255.0).astype(jnp.int32)
                word = word | (y << (8 * r))
            outbuf[bi * WPB + wr, pl.ds(cg * L, L)] = word

        if bi == BPB - 1:
            pltpu.async_copy(outbuf, out_dst(blk), sout)

    pltpu.make_async_copy(outbuf, out_dst(NB // BPB - 1), sout).wait()


def kernel(x, denominator_element_table, numerator_table):
    dtab = denominator_element_table
    thr = []
    for m in range(1, 8):
        ge = dtab >= m
        thr.append(jnp.where(ge.any(), jnp.argmax(ge), 256).astype(jnp.int32))
    den = _den_tc(x, jnp.stack(thr))
    return _main_kernel(x, numerator_table, den)


# speedup vs baseline: 1.1940x; 1.1940x over previous
import functools

import jax
import jax.numpy as jnp
from jax import lax
from jax.experimental import pallas as pl
from jax.experimental.pallas import tpu as pltpu
from jax.experimental.pallas import tpu_sc as plsc

NC = 2
NS = 16
L = 16
NW = NC * NS

N = 4096
C = 4096
RPW = N // NW
RB = 8
NB = RPW // RB

_CP = pltpu.CompilerParams(needs_layout_passes=False)


def _worker_id():
    return lax.axis_index("s") * NC + lax.axis_index("c")


_mesh = plsc.VectorSubcoreMesh(core_axis_name="c", subcore_axis_name="s")


N_TC = 2048
N_SC = N - N_TC
RPW_A = N_SC // NW
HC = C // 2
_RBLK = 256


def _den_tc_body(thr_ref, x_ref, den_ref):
    x = x_ref[...]
    acc = jnp.zeros(x.shape, jnp.int32)
    for m in range(7):
        acc += (x >= thr_ref[m]).astype(jnp.int32)
    den_ref[...] = jnp.sum(acc, axis=1).astype(jnp.float32)


def _den_tc(x, thr):
    return pl.pallas_call(
        _den_tc_body,
        grid=(N_TC // _RBLK,),
        in_specs=[
            pl.BlockSpec(memory_space=pltpu.SMEM),
            pl.BlockSpec((_RBLK, C), lambda i: (i, 0)),
        ],
        out_specs=pl.BlockSpec((_RBLK,), lambda i: (i,)),
        out_shape=jax.ShapeDtypeStruct((N_TC,), jnp.float32),
    )(thr, x)


@functools.partial(
    pl.kernel,
    out_type=jax.ShapeDtypeStruct((N_SC,), jnp.float32),
    mesh=_mesh,
    compiler_params=_CP,
    scratch_types=[
        pltpu.VMEM((2, L, HC), jnp.int32),
        pltpu.VMEM((256,), jnp.int32),
        pltpu.VMEM((256 * L,), jnp.int32),
        pltpu.VMEM((RPW_A,), jnp.float32),
        pltpu.SemaphoreType.DMA,
        pltpu.SemaphoreType.DMA,
    ],
)
def _den_sc(x_hbm, dtab_hbm, den_hbm, xbuf, dtab, dtab_rep, denout,
            sem0, sem1):
    wid = _worker_id()
    base_row = N_TC + wid * RPW_A
    pltpu.sync_copy(dtab_hbm, dtab)
    lanes = lax.iota(jnp.int32, L)
    sems = (sem0, sem1)

    @plsc.parallel_loop(0, 256, unroll=4)
    def _(v):
        dtab_rep[pl.ds(v * L, L)] = plsc.load_gather(
            dtab, [jnp.full((L,), v, jnp.int32)])

    def chunk_src(g, ch):
        return x_hbm.at[pl.ds(base_row + g * L, L), pl.ds(ch * HC, HC)]

    pltpu.async_copy(chunk_src(0, 0), xbuf.at[0], sem0)

    def grp_body(g, _):
        acc_g = jnp.zeros((L,), jnp.int32)
        for ch in (0, 1):
            pltpu.make_async_copy(chunk_src(g, ch), xbuf.at[ch],
                                  sems[ch]).wait()
            if ch == 0:
                pltpu.async_copy(chunk_src(g, 1), xbuf.at[1], sem1)
            else:
                @pl.when(g + 1 < RPW_A // L)
                def _():
                    pltpu.async_copy(chunk_src(g + 1, 0), xbuf.at[0], sem0)

            @plsc.parallel_loop(0, HC, unroll=8, carry=acc_g)
            def acc_g(c, acc_in):
                cvec = (lanes + c) & (HC - 1)
                xv = plsc.load_gather(xbuf.at[ch], [lanes, cvec])
                return acc_in + plsc.load_gather(dtab_rep, [(xv << 4) | lanes])

        denout[pl.ds(g * L, L)] = acc_g.astype(jnp.float32)
        return 0

    lax.fori_loop(0, RPW_A // L, grp_body, 0)
    pltpu.sync_copy(denout, den_hbm.at[pl.ds(wid * RPW_A, RPW_A)])


WPB = RB // 4
BLK = 8
BPB = BLK // WPB


@functools.partial(
    pl.kernel,
    out_type=jax.ShapeDtypeStruct((N, C), jnp.uint8),
    mesh=_mesh,
    compiler_params=_CP,
    scratch_types=[
        pltpu.VMEM((2, RB, C), jnp.int32),
        pltpu.VMEM((256,), jnp.int32),
        pltpu.VMEM((256 * L,), jnp.float32),
        pltpu.VMEM((N,), jnp.float32),
        pltpu.VMEM((BLK, C), jnp.int32),
        pltpu.SemaphoreType.DMA,
        pltpu.SemaphoreType.DMA,
        pltpu.SemaphoreType.DMA,
    ],
)
def _main_kernel(x_hbm, ntab_hbm, denlo_hbm, denhi_hbm, out_hbm,
                 xbuf, ntab, ntab_f, den, outbuf,
                 sin0, sin1, sout):
    wid = _worker_id()
    base_row = wid * RPW
    wout = out_hbm.bitcast(jnp.int32)
    base_wr = wid * (RPW // 4)
    pltpu.sync_copy(ntab_hbm, ntab)
    pltpu.sync_copy(denlo_hbm, den.at[pl.ds(0, N_TC)])
    pltpu.sync_copy(denhi_hbm, den.at[pl.ds(N_TC, N_SC)])
    sin = (sin0, sin1)
    lanes = lax.iota(jnp.int32, L)

    @plsc.parallel_loop(0, 256, unroll=4)
    def _(v):
        nv = plsc.load_gather(ntab, [jnp.full((L,), v, jnp.int32)])
        ntab_f[pl.ds(v * L, L)] = nv.astype(jnp.float32)

    def in_src(b):
        return x_hbm.at[pl.ds(base_row + b * RB, RB)]

    def out_dst(blk):
        return wout.at[pl.ds(base_wr + blk * BLK, BLK)]

    pltpu.async_copy(in_src(0), xbuf.at[0], sin0)

    for b in range(NB):
        s = b & 1
        blk, bi = divmod(b, BPB)
        pltpu.make_async_copy(in_src(b), xbuf.at[s], sin[s]).wait()
        if b + 1 < NB:
            pltpu.async_copy(in_src(b + 1), xbuf.at[1 - s], sin[1 - s])
        if bi == 0 and blk > 0:
            pltpu.make_async_copy(outbuf, out_dst(blk - 1), sout).wait()
        xb = xbuf.at[s]

        @plsc.parallel_loop(0, WPB * (C // L), unroll=2)
        def _(t):
            wr = t >> 8
            cg = t & (C // L - 1)
            dv = den[pl.ds(cg * L, L)]
            word = jnp.zeros((L,), jnp.int32)
            for r in range(4):
                xv = xb[wr * 4 + r, pl.ds(cg * L, L)]
                num = plsc.load_gather(ntab_f, [(xv << 4) | lanes])
                y = jnp.minimum(num / dv, 255.0).astype(jnp.int32)
                word = word | (y << (8 * r))
            outbuf[bi * WPB + wr, pl.ds(cg * L, L)] = word

        if bi == BPB - 1:
            pltpu.async_copy(outbuf, out_dst(blk), sout)

    pltpu.make_async_copy(outbuf, out_dst(NB // BPB - 1), sout).wait()


def kernel(x, denominator_element_table, numerator_table):
    dtab = denominator_element_table
    thr = []
    for m in range(1, 8):
        ge = dtab >= m
        thr.append(jnp.where(ge.any(), jnp.argmax(ge), 256).astype(jnp.int32))
    den_lo = _den_tc(x, jnp.stack(thr))
    den_hi = _den_sc(x, dtab)
    return _main_kernel(x, numerator_table, den_lo, den_hi)


# speedup vs baseline: 1.2224x; 1.0238x over previous
import functools

import jax
import jax.numpy as jnp
from jax import lax
from jax.experimental import pallas as pl
from jax.experimental.pallas import tpu as pltpu
from jax.experimental.pallas import tpu_sc as plsc

NC = 2
NS = 16
L = 16
NW = NC * NS

N = 4096
C = 4096
RPW = N // NW
RB = 8
NB = RPW // RB

_CP = pltpu.CompilerParams(needs_layout_passes=False)


def _worker_id():
    return lax.axis_index("s") * NC + lax.axis_index("c")


_mesh = plsc.VectorSubcoreMesh(core_axis_name="c", subcore_axis_name="s")


N_TC = 2048
N_SC = N - N_TC
RPW_A = N_SC // NW
HC = C // 2
_RBLK = 256


def _den_tc_body(thr_ref, x_ref, den_ref):
    x = x_ref[...]
    acc = jnp.zeros(x.shape, jnp.int32)
    for m in range(7):
        acc += (x >= thr_ref[m]).astype(jnp.int32)
    den_ref[...] = jnp.sum(acc, axis=1).astype(jnp.float32)


def _den_tc(x, thr):
    return pl.pallas_call(
        _den_tc_body,
        grid=(N_TC // _RBLK,),
        in_specs=[
            pl.BlockSpec(memory_space=pltpu.SMEM),
            pl.BlockSpec((_RBLK, C), lambda i: (i, 0)),
        ],
        out_specs=pl.BlockSpec((_RBLK,), lambda i: (i,)),
        out_shape=jax.ShapeDtypeStruct((N_TC,), jnp.float32),
    )(thr, x)


@functools.partial(
    pl.kernel,
    out_type=jax.ShapeDtypeStruct((N_SC,), jnp.float32),
    mesh=_mesh,
    compiler_params=_CP,
    scratch_types=[
        pltpu.VMEM((2, L, HC), jnp.int32),
        pltpu.VMEM((256,), jnp.int32),
        pltpu.VMEM((256 * L,), jnp.int32),
        pltpu.VMEM((RPW_A,), jnp.float32),
        pltpu.SemaphoreType.DMA,
        pltpu.SemaphoreType.DMA,
    ],
)
def _den_sc(x_hbm, dtab_hbm, den_hbm, xbuf, dtab, dtab_rep, denout,
            sem0, sem1):
    wid = _worker_id()
    base_row = N_TC + wid * RPW_A
    pltpu.sync_copy(dtab_hbm, dtab)
    lanes = lax.iota(jnp.int32, L)
    sems = (sem0, sem1)

    @plsc.parallel_loop(0, 256, unroll=4)
    def _(v):
        dtab_rep[pl.ds(v * L, L)] = plsc.load_gather(
            dtab, [jnp.full((L,), v, jnp.int32)])

    def chunk_src(g, ch):
        return x_hbm.at[pl.ds(base_row + g * L, L), pl.ds(ch * HC, HC)]

    pltpu.async_copy(chunk_src(0, 0), xbuf.at[0], sem0)

    def grp_body(g, _):
        acc_g = jnp.zeros((L,), jnp.int32)
        for ch in (0, 1):
            pltpu.make_async_copy(chunk_src(g, ch), xbuf.at[ch],
                                  sems[ch]).wait()
            if ch == 0:
                pltpu.async_copy(chunk_src(g, 1), xbuf.at[1], sem1)
            else:
                @pl.when(g + 1 < RPW_A // L)
                def _():
                    pltpu.async_copy(chunk_src(g + 1, 0), xbuf.at[0], sem0)

            @plsc.parallel_loop(0, HC, unroll=8, carry=acc_g)
            def acc_g(c, acc_in):
                cvec = (lanes + c) & (HC - 1)
                xv = plsc.load_gather(xbuf.at[ch], [lanes, cvec])
                return acc_in + plsc.load_gather(dtab_rep, [(xv << 4) | lanes])

        denout[pl.ds(g * L, L)] = acc_g.astype(jnp.float32)
        return 0

    lax.fori_loop(0, RPW_A // L, grp_body, 0)
    pltpu.sync_copy(denout, den_hbm.at[pl.ds(wid * RPW_A, RPW_A)])


WPB = RB // 4
BLK = 8
BPB = BLK // WPB


@functools.partial(
    pl.kernel,
    out_type=jax.ShapeDtypeStruct((N, C), jnp.uint8),
    mesh=_mesh,
    compiler_params=_CP,
    scratch_types=[
        pltpu.VMEM((2, RB, C), jnp.int32),
        pltpu.VMEM((256,), jnp.int32),
        pltpu.VMEM((256 * L,), jnp.float32),
        pltpu.VMEM((N,), jnp.float32),
        pltpu.VMEM((BLK, C), jnp.int32),
        pltpu.SemaphoreType.DMA,
        pltpu.SemaphoreType.DMA,
        pltpu.SemaphoreType.DMA,
    ],
)
def _main_kernel(x_hbm, ntab_hbm, denlo_hbm, denhi_hbm, out_hbm,
                 xbuf, ntab, ntab_f, den, outbuf,
                 sin0, sin1, sout):
    wid = _worker_id()
    base_row = wid * RPW
    wout = out_hbm.bitcast(jnp.int32)
    base_wr = wid * (RPW // 4)
    pltpu.sync_copy(ntab_hbm, ntab)
    pltpu.sync_copy(denlo_hbm, den.at[pl.ds(0, N_TC)])
    pltpu.sync_copy(denhi_hbm, den.at[pl.ds(N_TC, N_SC)])
    sin = (sin0, sin1)
    lanes = lax.iota(jnp.int32, L)

    @plsc.parallel_loop(0, 256, unroll=4)
    def _(v):
        nv = plsc.load_gather(ntab, [jnp.full((L,), v, jnp.int32)])
        ntab_f[pl.ds(v * L, L)] = nv.astype(jnp.float32)

    def in_src(b):
        return x_hbm.at[pl.ds(base_row + b * RB, RB)]

    def out_dst(blk):
        return wout.at[pl.ds(base_wr + blk * BLK, BLK)]

    pltpu.async_copy(in_src(0), xbuf.at[0], sin0)

    for b in range(NB):
        s = b & 1
        blk, bi = divmod(b, BPB)
        pltpu.make_async_copy(in_src(b), xbuf.at[s], sin[s]).wait()
        if b + 1 < NB:
            pltpu.async_copy(in_src(b + 1), xbuf.at[1 - s], sin[1 - s])
        if bi == 0 and blk > 0:
            pltpu.make_async_copy(outbuf, out_dst(blk - 1), sout).wait()
        xb = xbuf.at[s]

        @plsc.parallel_loop(0, WPB * (C // L), unroll=2)
        def _(t):
            wr = t >> 8
            cg = t & (C // L - 1)
            dv = den[pl.ds(cg * L, L)]
            word = jnp.zeros((L,), jnp.int32)
            for r in range(4):
                xv = xb[wr * 4 + r, pl.ds(cg * L, L)]
                num = plsc.load_gather(ntab_f, [(xv << 4) | lanes])
                y = jnp.minimum(num / dv, 255.0).astype(jnp.int32)
                word = word | (y << (8 * r))
            outbuf[bi * WPB + wr, pl.ds(cg * L, L)] = word

        if bi == BPB - 1:
            pltpu.async_copy(outbuf, out_dst(blk), sout)

    pltpu.make_async_copy(outbuf, out_dst(NB // BPB - 1), sout).wait()


def kernel(x, denominator_element_table, numerator_table):
    dtab = denominator_element_table
    thr = jnp.sum((dtab[None, :] < jnp.arange(1, 8)[:, None]),
                  axis=1, dtype=jnp.int32)
    den_lo = _den_tc(x, thr)
    den_hi = _den_sc(x, dtab)
    return _main_kernel(x, numerator_table, den_lo, den_hi)


# speedup vs baseline: 1.2655x; 1.0353x over previous
import functools

import jax
import jax.numpy as jnp
from jax import lax
from jax.experimental import pallas as pl
from jax.experimental.pallas import tpu as pltpu
from jax.experimental.pallas import tpu_sc as plsc

NC = 2
NS = 16
L = 16
NW = NC * NS

N = 4096
C = 4096
RPW = N // NW
RB = 8
NB = RPW // RB

_CP = pltpu.CompilerParams(needs_layout_passes=False)


def _worker_id():
    return lax.axis_index("s") * NC + lax.axis_index("c")


_mesh = plsc.VectorSubcoreMesh(core_axis_name="c", subcore_axis_name="s")


N_TC = 2048
N_SC = N - N_TC
RPW_A = N_SC // NW
HC = C // 2
_RBLK = 256


def _den_tc_body(thr_ref, x_ref, den_ref):
    x = x_ref[...]
    acc = jnp.zeros(x.shape, jnp.int32)
    for m in range(7):
        acc += (x >= thr_ref[m]).astype(jnp.int32)
    den_ref[...] = jnp.sum(acc, axis=1).astype(jnp.float32)


def _den_tc(x, thr):
    return pl.pallas_call(
        _den_tc_body,
        grid=(N_TC // _RBLK,),
        in_specs=[
            pl.BlockSpec(memory_space=pltpu.SMEM),
            pl.BlockSpec((_RBLK, C), lambda i: (i, 0)),
        ],
        out_specs=pl.BlockSpec((_RBLK,), lambda i: (i,)),
        out_shape=jax.ShapeDtypeStruct((N_TC,), jnp.float32),
    )(thr, x)


@functools.partial(
    pl.kernel,
    out_type=jax.ShapeDtypeStruct((N_SC,), jnp.float32),
    mesh=_mesh,
    compiler_params=_CP,
    scratch_types=[
        pltpu.VMEM((2, L, HC), jnp.int32),
        pltpu.VMEM((256,), jnp.int32),
        pltpu.VMEM((256 * L,), jnp.int32),
        pltpu.VMEM((RPW_A,), jnp.float32),
        pltpu.SemaphoreType.DMA,
        pltpu.SemaphoreType.DMA,
    ],
)
def _den_sc(x_hbm, dtab_hbm, den_hbm, xbuf, dtab, dtab_rep, denout,
            sem0, sem1):
    wid = _worker_id()
    base_row = N_TC + wid * RPW_A
    pltpu.sync_copy(dtab_hbm, dtab)
    lanes = lax.iota(jnp.int32, L)
    sems = (sem0, sem1)

    @plsc.parallel_loop(0, 256, unroll=4)
    def _(v):
        dtab_rep[pl.ds(v * L, L)] = plsc.load_gather(
            dtab, [jnp.full((L,), v, jnp.int32)])

    def chunk_src(g, ch):
        return x_hbm.at[pl.ds(base_row + g * L, L), pl.ds(ch * HC, HC)]

    pltpu.async_copy(chunk_src(0, 0), xbuf.at[0], sem0)

    def grp_body(g, _):
        acc_g = jnp.zeros((L,), jnp.int32)
        for ch in (0, 1):
            pltpu.make_async_copy(chunk_src(g, ch), xbuf.at[ch],
                                  sems[ch]).wait()
            if ch == 0:
                pltpu.async_copy(chunk_src(g, 1), xbuf.at[1], sem1)
            else:
                @pl.when(g + 1 < RPW_A // L)
                def _():
                    pltpu.async_copy(chunk_src(g + 1, 0), xbuf.at[0], sem0)

            @plsc.parallel_loop(0, HC, unroll=8, carry=acc_g)
            def acc_g(c, acc_in):
                cvec = (lanes + c) & (HC - 1)
                xv = plsc.load_gather(xbuf.at[ch], [lanes, cvec])
                return acc_in + plsc.load_gather(dtab_rep, [(xv << 4) | lanes])

        denout[pl.ds(g * L, L)] = acc_g.astype(jnp.float32)
        return 0

    lax.fori_loop(0, RPW_A // L, grp_body, 0)
    pltpu.sync_copy(denout, den_hbm.at[pl.ds(wid * RPW_A, RPW_A)])


WPB = RB // 4
BLK = 8
BPB = BLK // WPB


@functools.partial(
    pl.kernel,
    out_type=jax.ShapeDtypeStruct((N, C), jnp.uint8),
    mesh=_mesh,
    compiler_params=_CP,
    scratch_types=[
        pltpu.VMEM((2, RB, C), jnp.int32),
        pltpu.VMEM((256,), jnp.int32),
        pltpu.VMEM((256 * L,), jnp.float32),
        pltpu.VMEM((N,), jnp.float32),
        pltpu.VMEM((BLK, C), jnp.int32),
        pltpu.SemaphoreType.DMA,
        pltpu.SemaphoreType.DMA,
        pltpu.SemaphoreType.DMA,
    ],
)
def _main_kernel(x_hbm, ntab_hbm, denlo_hbm, denhi_hbm, out_hbm,
                 xbuf, ntab, ntab_f, den, outbuf,
                 sin0, sin1, sout):
    wid = _worker_id()
    base_row = wid * RPW
    wout = out_hbm.bitcast(jnp.int32)
    base_wr = wid * (RPW // 4)
    pltpu.sync_copy(ntab_hbm, ntab)
    pltpu.sync_copy(denlo_hbm, den.at[pl.ds(0, N_TC)])
    pltpu.sync_copy(denhi_hbm, den.at[pl.ds(N_TC, N_SC)])
    sin = (sin0, sin1)
    lanes = lax.iota(jnp.int32, L)

    @plsc.parallel_loop(0, 256, unroll=4)
    def _(v):
        nv = plsc.load_gather(ntab, [jnp.full((L,), v, jnp.int32)])
        ntab_f[pl.ds(v * L, L)] = nv.astype(jnp.float32)

    def in_src(b):
        return x_hbm.at[pl.ds(base_row + b * RB, RB)]

    def out_dst(blk):
        return wout.at[pl.ds(base_wr + blk * BLK, BLK)]

    pltpu.async_copy(in_src(0), xbuf.at[0], sin0)

    for b in range(NB):
        s = b & 1
        blk, bi = divmod(b, BPB)
        pltpu.make_async_copy(in_src(b), xbuf.at[s], sin[s]).wait()
        if b + 1 < NB:
            pltpu.async_copy(in_src(b + 1), xbuf.at[1 - s], sin[1 - s])
        if bi == 0 and blk > 0:
            pltpu.make_async_copy(outbuf, out_dst(blk - 1), sout).wait()
        xb = xbuf.at[s]

        @plsc.parallel_loop(0, WPB * (C // L), unroll=4)
        def _(t):
            wr = t >> 8
            cg = t & (C // L - 1)
            dv = den[pl.ds(cg * L, L)]
            word = jnp.zeros((L,), jnp.int32)
            for r in range(4):
                xv = xb[wr * 4 + r, pl.ds(cg * L, L)]
                num = plsc.load_gather(ntab_f, [(xv << 4) | lanes])
                y = jnp.minimum(num / dv, 255.0).astype(jnp.int32)
                word = word | (y << (8 * r))
            outbuf[bi * WPB + wr, pl.ds(cg * L, L)] = word

        if bi == BPB - 1:
            pltpu.async_copy(outbuf, out_dst(blk), sout)

    pltpu.make_async_copy(outbuf, out_dst(NB // BPB - 1), sout).wait()


def kernel(x, denominator_element_table, numerator_table):
    dtab = denominator_element_table
    thr = jnp.sum((dtab[None, :] < jnp.arange(1, 8)[:, None]),
                  axis=1, dtype=jnp.int32)
    den_lo = _den_tc(x, thr)
    den_hi = _den_sc(x, dtab)
    return _main_kernel(x, numerator_table, den_lo, den_hi)


# speedup vs baseline: 1.2896x; 1.0191x over previous
import functools

import jax
import jax.numpy as jnp
from jax import lax
from jax.experimental import pallas as pl
from jax.experimental.pallas import tpu as pltpu
from jax.experimental.pallas import tpu_sc as plsc

NC = 2
NS = 16
L = 16
NW = NC * NS

N = 4096
C = 4096
RPW = N // NW
RB = 8
NB = RPW // RB

_CP = pltpu.CompilerParams(needs_layout_passes=False)


def _worker_id():
    return lax.axis_index("s") * NC + lax.axis_index("c")


_mesh = plsc.VectorSubcoreMesh(core_axis_name="c", subcore_axis_name="s")


N_TC = 2048
N_SC = N - N_TC
RPW_A = N_SC // NW
HC = C // 2
_RBLK = 256


def _den_tc_body(thr_ref, x_ref, den_ref):
    x = x_ref[...]
    acc = jnp.zeros(x.shape, jnp.int32)
    for m in range(7):
        acc += (x >= thr_ref[m]).astype(jnp.int32)
    den_ref[...] = jnp.sum(acc, axis=1).astype(jnp.float32)


def _den_tc(x, thr):
    return pl.pallas_call(
        _den_tc_body,
        grid=(N_TC // _RBLK,),
        in_specs=[
            pl.BlockSpec(memory_space=pltpu.SMEM),
            pl.BlockSpec((_RBLK, C), lambda i: (i, 0)),
        ],
        out_specs=pl.BlockSpec((_RBLK,), lambda i: (i,)),
        out_shape=jax.ShapeDtypeStruct((N_TC,), jnp.float32),
    )(thr, x)


@functools.partial(
    pl.kernel,
    out_type=jax.ShapeDtypeStruct((N_SC,), jnp.float32),
    mesh=_mesh,
    compiler_params=_CP,
    scratch_types=[
        pltpu.VMEM((2, L, HC), jnp.int32),
        pltpu.VMEM((256,), jnp.int32),
        pltpu.VMEM((256 * L,), jnp.int32),
        pltpu.VMEM((RPW_A,), jnp.float32),
        pltpu.SemaphoreType.DMA,
        pltpu.SemaphoreType.DMA,
    ],
)
def _den_sc(x_hbm, dtab_hbm, den_hbm, xbuf, dtab, dtab_rep, denout,
            sem0, sem1):
    wid = _worker_id()
    base_row = N_TC + wid * RPW_A
    pltpu.sync_copy(dtab_hbm, dtab)
    lanes = lax.iota(jnp.int32, L)
    sems = (sem0, sem1)

    @plsc.parallel_loop(0, 256, unroll=4)
    def _(v):
        dtab_rep[pl.ds(v * L, L)] = plsc.load_gather(
            dtab, [jnp.full((L,), v, jnp.int32)])

    def chunk_src(g, ch):
        return x_hbm.at[pl.ds(base_row + g * L, L), pl.ds(ch * HC, HC)]

    pltpu.async_copy(chunk_src(0, 0), xbuf.at[0], sem0)

    def grp_body(g, _):
        acc_g = jnp.zeros((L,), jnp.int32)
        for ch in (0, 1):
            pltpu.make_async_copy(chunk_src(g, ch), xbuf.at[ch],
                                  sems[ch]).wait()
            if ch == 0:
                pltpu.async_copy(chunk_src(g, 1), xbuf.at[1], sem1)
            else:
                @pl.when(g + 1 < RPW_A // L)
                def _():
                    pltpu.async_copy(chunk_src(g + 1, 0), xbuf.at[0], sem0)

            @plsc.parallel_loop(0, HC, unroll=8, carry=acc_g)
            def acc_g(c, acc_in):
                cvec = (lanes + c) & (HC - 1)
                xv = plsc.load_gather(xbuf.at[ch], [lanes, cvec])
                return acc_in + plsc.load_gather(dtab_rep, [(xv << 4) | lanes])

        denout[pl.ds(g * L, L)] = acc_g.astype(jnp.float32)
        return 0

    lax.fori_loop(0, RPW_A // L, grp_body, 0)
    pltpu.sync_copy(denout, den_hbm.at[pl.ds(wid * RPW_A, RPW_A)])


WPB = RB // 4
BLK = 8
BPB = BLK // WPB


@functools.partial(
    pl.kernel,
    out_type=jax.ShapeDtypeStruct((N, C), jnp.uint8),
    mesh=_mesh,
    compiler_params=_CP,
    scratch_types=[
        pltpu.VMEM((2, RB, C), jnp.int32),
        pltpu.VMEM((256,), jnp.int32),
        pltpu.VMEM((256 * L,), jnp.float32),
        pltpu.VMEM((N,), jnp.float32),
        pltpu.VMEM((BLK, C), jnp.int32),
        pltpu.SemaphoreType.DMA,
        pltpu.SemaphoreType.DMA,
        pltpu.SemaphoreType.DMA,
    ],
)
def _main_kernel(x_hbm, ntab_hbm, denlo_hbm, denhi_hbm, out_hbm,
                 xbuf, ntab, ntab_f, den, outbuf,
                 sin0, sin1, sout):
    wid = _worker_id()
    base_row = wid * RPW
    wout = out_hbm.bitcast(jnp.int32)
    base_wr = wid * (RPW // 4)
    pltpu.sync_copy(ntab_hbm, ntab)
    pltpu.sync_copy(denlo_hbm, den.at[pl.ds(0, N_TC)])
    pltpu.sync_copy(denhi_hbm, den.at[pl.ds(N_TC, N_SC)])
    sin = (sin0, sin1)
    lanes = lax.iota(jnp.int32, L)

    @plsc.parallel_loop(0, 256, unroll=4)
    def _(v):
        nv = plsc.load_gather(ntab, [jnp.full((L,), v, jnp.int32)])
        ntab_f[pl.ds(v * L, L)] = nv.astype(jnp.float32)

    def in_src(b):
        return x_hbm.at[pl.ds(base_row + b * RB, RB)]

    def out_dst(blk):
        return wout.at[pl.ds(base_wr + blk * BLK, BLK)]

    pltpu.async_copy(in_src(0), xbuf.at[0], sin0)

    for b in range(NB):
        s = b & 1
        blk, bi = divmod(b, BPB)
        pltpu.make_async_copy(in_src(b), xbuf.at[s], sin[s]).wait()
        if b + 1 < NB:
            pltpu.async_copy(in_src(b + 1), xbuf.at[1 - s], sin[1 - s])
        if bi == 0 and blk > 0:
            pltpu.make_async_copy(outbuf, out_dst(blk - 1), sout).wait()
        xb = xbuf.at[s]

        @plsc.parallel_loop(0, WPB * (C // L), unroll=8)
        def _(t):
            wr = t >> 8
            cg = t & (C // L - 1)
            dv = den[pl.ds(cg * L, L)]
            word = jnp.zeros((L,), jnp.int32)
            for r in range(4):
                xv = xb[wr * 4 + r, pl.ds(cg * L, L)]
                num = plsc.load_gather(ntab_f, [(xv << 4) | lanes])
                y = jnp.minimum(num / dv, 255.0).astype(jnp.int32)
                word = word | (y << (8 * r))
            outbuf[bi * WPB + wr, pl.ds(cg * L, L)] = word

        if bi == BPB - 1:
            pltpu.async_copy(outbuf, out_dst(blk), sout)

    pltpu.make_async_copy(outbuf, out_dst(NB // BPB - 1), sout).wait()


def kernel(x, denominator_element_table, numerator_table):
    dtab = denominator_element_table
    thr = jnp.sum((dtab[None, :] < jnp.arange(1, 8)[:, None]),
                  axis=1, dtype=jnp.int32)
    den_lo = _den_tc(x, thr)
    den_hi = _den_sc(x, dtab)
    return _main_kernel(x, numerator_table, den_lo, den_hi)
